# Initial kernel scaffold; baseline (speedup 1.0000x reference)
import jax
import jax.numpy as jnp
from jax.experimental import pallas as pl


def kernel(x, table):
    raise NotImplementedError("write your pallas kernel here")


# speedup vs baseline: 3.8720x; 3.8720x over previous
import functools

import jax
import jax.numpy as jnp
from jax import lax
from jax.experimental import pallas as pl
from jax.experimental.pallas import tpu as pltpu
from jax.experimental.pallas import tpu_sc as plsc

_NC = 2
_NS = 16
_NW = _NC * _NS

_IDX_W = 128
_RPC = 4


def _norm_body(t_ref, o_ref):
    t = t_ref[...]
    y = jnp.maximum(t, 0.0)
    ss = jnp.sum(y * y, axis=-1, keepdims=True)
    nrm = jnp.maximum(jnp.sqrt(ss), 1e-12)
    o_ref[...] = y / nrm


def _normalize_table(table):
    v, d = table.shape
    blk = 2000
    assert v % blk == 0
    return pl.pallas_call(
        _norm_body,
        grid=(v // blk,),
        in_specs=[pl.BlockSpec((blk, d), lambda i: (i, 0))],
        out_specs=pl.BlockSpec((blk, d), lambda i: (i, 0)),
        out_shape=jax.ShapeDtypeStruct((v, d), jnp.float32),
    )(table)


def _make_sc_gather(v, d, b):
    assert b % (_NW * _IDX_W * _RPC) == 0
    rows_per_w = b // (_NW * _IDX_W)
    n_chunks = rows_per_w // _RPC
    mesh = plsc.VectorSubcoreMesh(core_axis_name="c", subcore_axis_name="s")

    @functools.partial(
        pl.kernel,
        out_type=jax.ShapeDtypeStruct((b, d), jnp.float32),
        mesh=mesh,
        compiler_params=pltpu.CompilerParams(use_tc_tiling_on_sc=False),
        scratch_types=[
            pltpu.VMEM((_RPC, _IDX_W), jnp.int32),
            pltpu.VMEM((_RPC * _IDX_W, d), jnp.float32),
            pltpu.SemaphoreType.DMA,
        ],
    )
    def gather_kernel(table_hbm, idx_hbm, out_hbm, idx_v, rows_v, sem):
        wid = lax.axis_index("s") * _NC + lax.axis_index("c")
        row0 = wid * rows_per_w

        def chunk(c, carry):
            r0 = row0 + c * _RPC
            pltpu.sync_copy(idx_hbm.at[pl.ds(r0, _RPC)], idx_v)
            cps = [
                pltpu.async_copy(
                    table_hbm.at[idx_v.at[j]],
                    rows_v.at[pl.ds(j * _IDX_W, _IDX_W)],
                    sem,
                )
                for j in range(_RPC)
            ]
            for cp in cps:
                cp.wait()
            pltpu.sync_copy(
                rows_v, out_hbm.at[pl.ds(r0 * _IDX_W, _RPC * _IDX_W)]
            )
            return carry

        lax.fori_loop(0, n_chunks, chunk, 0)

    return gather_kernel


def kernel(x, table):
    bsz, f = x.shape
    v, d = table.shape
    b = bsz * f
    table2 = _normalize_table(table)
    idx2d = x.reshape(b // _IDX_W, _IDX_W).astype(jnp.int32)
    out_flat = _make_sc_gather(v, d, b)(table2, idx2d)
    return out_flat.reshape(bsz, f, d)


# speedup vs baseline: 4.0568x; 1.0477x over previous
import functools

import jax
import jax.numpy as jnp
from jax import lax
from jax.experimental import pallas as pl
from jax.experimental.pallas import tpu as pltpu
from jax.experimental.pallas import tpu_sc as plsc

_NC = 2
_NS = 16
_NW = _NC * _NS

_IDX_W = 128
_RPC = 4


def _norm_body(t_ref, o_ref):
    t = t_ref[...]
    y = jnp.maximum(t, 0.0)
    ss = jnp.sum(y * y, axis=-1, keepdims=True)
    nrm = jnp.maximum(jnp.sqrt(ss), 1e-12)
    o_ref[...] = y / nrm


def _normalize_table(table):
    v, d = table.shape
    blk = 2000
    assert v % blk == 0
    return pl.pallas_call(
        _norm_body,
        grid=(v // blk,),
        in_specs=[pl.BlockSpec((blk, d), lambda i: (i, 0))],
        out_specs=pl.BlockSpec((blk, d), lambda i: (i, 0)),
        out_shape=jax.ShapeDtypeStruct((v, d), jnp.float32),
    )(table)


_NBUF = 2


def _make_sc_gather(v, d, b):
    assert b % (_NW * _IDX_W * _RPC) == 0
    rows_per_w = b // (_NW * _IDX_W)
    n_chunks = rows_per_w // _RPC
    assert n_chunks % _NBUF == 0
    mesh = plsc.VectorSubcoreMesh(core_axis_name="c", subcore_axis_name="s")
    ck = _RPC * _IDX_W

    @functools.partial(
        pl.kernel,
        out_type=jax.ShapeDtypeStruct((b, d), jnp.float32),
        mesh=mesh,
        compiler_params=pltpu.CompilerParams(use_tc_tiling_on_sc=False),
        scratch_types=[
            [pltpu.VMEM((_RPC, _IDX_W), jnp.int32)] * _NBUF,
            [pltpu.VMEM((ck, d), jnp.float32)] * _NBUF,
            [pltpu.SemaphoreType.DMA] * (3 * _NBUF),
        ],
    )
    def gather_kernel(table_hbm, idx_hbm, out_hbm, idx_bufs, row_bufs, sems):
        isems, gsems, ssems = sems[:_NBUF], sems[_NBUF:2 * _NBUF], sems[2 * _NBUF:]
        wid = lax.axis_index("s") * _NC + lax.axis_index("c")
        row0 = wid * rows_per_w

        def idx_start(c, bi):
            pltpu.async_copy(
                idx_hbm.at[pl.ds(row0 + c * _RPC, _RPC)], idx_bufs[bi], isems[bi]
            )

        for bi in range(_NBUF):
            idx_start(bi, bi)

        @pl.loop(0, n_chunks, step=_NBUF)
        def _outer(g):
            for bi in range(_NBUF):
                c = g + bi

                @pl.when(c >= _NBUF)
                def _():
                    pltpu.make_async_copy(
                        row_bufs[bi], out_hbm.at[pl.ds(0, ck)], ssems[bi]
                    ).wait()

                pltpu.make_async_copy(
                    idx_hbm.at[pl.ds(0, _RPC)], idx_bufs[bi], isems[bi]
                ).wait()
                cps = [
                    pltpu.async_copy(
                        table_hbm.at[idx_bufs[bi].at[j]],
                        row_bufs[bi].at[pl.ds(j * _IDX_W, _IDX_W)],
                        gsems[bi],
                    )
                    for j in range(_RPC)
                ]
                for cp in cps:
                    cp.wait()

                @pl.when(c + _NBUF < n_chunks)
                def _():
                    idx_start(c + _NBUF, bi)

                pltpu.async_copy(
                    row_bufs[bi],
                    out_hbm.at[pl.ds((row0 + c * _RPC) * _IDX_W, ck)],
                    ssems[bi],
                )

        for bi in range(_NBUF):
            pltpu.make_async_copy(
                row_bufs[bi], out_hbm.at[pl.ds(0, ck)], ssems[bi]
            ).wait()

    return gather_kernel


def kernel(x, table):
    bsz, f = x.shape
    v, d = table.shape
    b = bsz * f
    table2 = _normalize_table(table)
    idx2d = x.reshape(b // _IDX_W, _IDX_W).astype(jnp.int32)
    out_flat = _make_sc_gather(v, d, b)(table2, idx2d)
    return out_flat.reshape(bsz, f, d)
